# baseline (speedup 1.0000x reference)
import jax
import jax.numpy as jnp
from jax import lax
from jax.experimental import pallas as pl
from jax.experimental.pallas import tpu as pltpu
from jax.experimental.pallas import tpu_sc as plsc

N = 10000
E = 160000
D_IN = 256
D_OUT = 768
NC = 2
DCH = D_OUT // NC
NGRP = DCH // 32
NW = 32
NPW = 320
NP = NW * NPW
RB = 1024
EB = 2000
NB = E // EB
G = 64
GQ = G // 16
C_CAP = 8192
LIST_SZ = C_CAP + EB + 16
DUMMY = NPW


def _lin_body(x_ref, w_ref, b_ref, *y_refs):
    y = lax.dot_general(x_ref[...], w_ref[...], (((1,), (1,)), ((), ())),
                        preferred_element_type=jnp.float32)
    y = jnp.maximum(y + b_ref[0, :], 0.0)
    for c in range(NC):
        y_refs[c][...] = y[:, c * DCH:(c + 1) * DCH].astype(jnp.bfloat16)


def _msg_linear(x_pad, W_lin, b2):
    return pl.pallas_call(
        _lin_body,
        grid=(NP // RB,),
        in_specs=[
            pl.BlockSpec((RB, D_IN), lambda i: (i, 0)),
            pl.BlockSpec((D_OUT, D_IN), lambda i: (0, 0)),
            pl.BlockSpec((1, D_OUT), lambda i: (0, 0)),
        ],
        out_specs=[pl.BlockSpec((RB, DCH), lambda i: (i, 0))] * NC,
        out_shape=[jax.ShapeDtypeStruct((NP, DCH), jnp.bfloat16)] * NC,
    )(x_pad, W_lin, b2)


def _segmax_body(y0, y1, src_hbm, dst_hbm,
                 o0, o1,
                 acc, lst, srcb0, dstb0, srcb1, dstb1, rb0, rb1, ix0, ix1,
                 shs, shd, sg0, sg1, ss0, ss1):
    sid = lax.axis_index("s")
    wid = sid * 2 + lax.axis_index("c")
    base = wid * NPW
    iota = lax.broadcasted_iota(jnp.int32, (16,), 0)

    @pl.when(sid == 0)
    def _():
        pltpu.sync_copy(src_hbm, shs)
        pltpu.sync_copy(dst_hbm, shd)
    plsc.subcore_barrier()

    def zero_body(i, _):
        lst[pl.ds(i * 16, 16)] = jnp.full((16,), DUMMY, jnp.int32)
        return 0
    lax.fori_loop(0, LIST_SZ // 16, zero_body, 0)

    def stage_start(b, sb, db, sem):
        off = b * EB
        pltpu.make_async_copy(shs.at[pl.ds(off, EB)], sb, sem).start()
        pltpu.make_async_copy(shd.at[pl.ds(off, EB)], db, sem).start()

    def stage_wait(b, sb, db, sem):
        off = b * EB
        pltpu.make_async_copy(shs.at[pl.ds(off, EB)], sb, sem).wait()
        pltpu.make_async_copy(shd.at[pl.ds(off, EB)], db, sem).wait()

    def filt(sb, db, lbase, cnt):
        def fb(i, cn):
            d = db[pl.ds(i * 16, 16)]
            s = sb[pl.ds(i * 16, 16)]
            ldl = d - base
            m = (ldl >= 0) & (ldl < NPW)
            mi = jnp.where(m, 1, 0).astype(jnp.int32)
            incl = plsc.cumsum(mi)
            pos = (lbase + cn - 1) + incl
            plsc.store_scatter(lst, [pos], s * 512 + ldl, mask=m)
            return cn + incl[15]
        return lax.fori_loop(0, EB // 16, fb, cnt)

    def pad16(lbase, cnt):
        pos = lbase + cnt + iota
        plsc.store_scatter(lst, [pos], jnp.full((16,), DUMMY, jnp.int32))

    stage_start(0, srcb0, dstb0, ss0)

    def p1_body(p, carry):
        cnt, spill = carry

        def half(b, sb, db, sem, cnt, spill):
            newcnt = filt(sb, db, 0, cnt)
            ok = newcnt <= C_CAP
            cnt = jnp.where(ok, newcnt, cnt)
            spill = jnp.where((~ok) & (spill == NB), b, spill)
            return cnt, spill

        b0 = p * 2
        stage_start(b0 + 1, srcb1, dstb1, ss1)
        stage_wait(b0, srcb0, dstb0, ss0)
        cnt, spill = half(b0, srcb0, dstb0, ss0, cnt, spill)

        @pl.when(b0 + 2 < NB)
        def _():
            stage_start(b0 + 2, srcb0, dstb0, ss0)
        stage_wait(b0 + 1, srcb1, dstb1, ss1)
        cnt, spill = half(b0 + 1, srcb1, dstb1, ss1, cnt, spill)
        return cnt, spill

    cnt, spill = lax.fori_loop(0, NB // 2, p1_body,
                               (jnp.int32(0), jnp.int32(NB)))
    pad16(0, cnt)
    nq_main = (cnt + 15) // 16

    def run_waves(yc, ebase, nq):
        nw = (nq + GQ - 1) // GQ

        def issue(w, rbk, ixk, semk):
            def ub(i, _):
                pv = lst[pl.ds(ebase + w * G + i * 16, 16)]
                ixk[pl.ds(i * 16, 16)] = jnp.right_shift(pv, 9)
                return 0
            lax.fori_loop(0, G // 16, ub, 0)
            pltpu.make_async_copy(yc.at[ixk], rbk, semk).start()

        def wait_g(yc, rbk, ixk, semk):
            pltpu.make_async_copy(yc.at[ixk], rbk, semk).wait()

        def process(w, rbk):
            ne = jnp.minimum(nq - w * GQ, GQ) * 16

            def eb_body(j, _):
                pk = lst[pl.ds(ebase + w * G + j, 16)][0]
                ld = lax.rem(pk, 512)

                @plsc.parallel_loop(0, NGRP, unroll=NGRP)
                def cb(g, ld=ld, j=j):
                    sl = pl.ds(g * 32, 32)
                    acc[ld, sl] = jnp.maximum(acc[ld, sl], rbk[j, sl])
                return 0
            lax.fori_loop(0, ne, eb_body, 0)

        @pl.when(nw > 0)
        def _():
            issue(jnp.int32(0), rb0, ix0, sg0)

        def pair_body(p2, _):
            w0 = p2 * 2
            w1 = w0 + 1

            @pl.when(w1 < nw)
            def _():
                issue(w1, rb1, ix1, sg1)
            wait_g(yc, rb0, ix0, sg0)
            process(w0, rb0)

            @pl.when(w0 + 2 < nw)
            def _():
                issue(w0 + 2, rb0, ix0, sg0)

            @pl.when(w1 < nw)
            def _():
                wait_g(yc, rb1, ix1, sg1)
                process(w1, rb1)
            return 0
        lax.fori_loop(0, (nw + 1) // 2, pair_body, 0)

    for yc, oc in ((y0, o0), (y1, o1)):
        pltpu.sync_copy(yc.at[pl.ds(base, NPW)], acc.at[pl.ds(0, NPW)])
        run_waves(yc, 0, nq_main)

        def spill_body(b, _, yc=yc):
            off = b * EB
            pltpu.sync_copy(src_hbm.at[pl.ds(off, EB)], srcb0)
            pltpu.sync_copy(dst_hbm.at[pl.ds(off, EB)], dstb0)
            tcnt = filt(srcb0, dstb0, C_CAP, jnp.int32(0))
            pad16(C_CAP, tcnt)
            run_waves(yc, C_CAP, (tcnt + 15) // 16)
            return 0
        lax.fori_loop(spill, NB, spill_body, 0)

        pltpu.sync_copy(acc.at[pl.ds(0, NPW)], oc.at[pl.ds(base, NPW)])


def _segment_max(Y4, src, dst):
    mesh = plsc.VectorSubcoreMesh(core_axis_name="c", subcore_axis_name="s")
    f = pl.kernel(
        _segmax_body,
        out_type=[jax.ShapeDtypeStruct((NP, DCH), jnp.bfloat16)] * NC,
        mesh=mesh,
        compiler_params=pltpu.CompilerParams(use_tc_tiling_on_sc=False,
                                             needs_layout_passes=False),
        scratch_types=[
            pltpu.VMEM((NPW + 1, DCH), jnp.bfloat16),
            pltpu.VMEM((LIST_SZ,), jnp.int32),
            pltpu.VMEM((EB,), jnp.int32),
            pltpu.VMEM((EB,), jnp.int32),
            pltpu.VMEM((EB,), jnp.int32),
            pltpu.VMEM((EB,), jnp.int32),
            pltpu.VMEM((G, DCH), jnp.bfloat16),
            pltpu.VMEM((G, DCH), jnp.bfloat16),
            pltpu.VMEM((G,), jnp.int32),
            pltpu.VMEM((G,), jnp.int32),
            pltpu.VMEM_SHARED((E,), jnp.int32),
            pltpu.VMEM_SHARED((E,), jnp.int32),
            pltpu.SemaphoreType.DMA,
            pltpu.SemaphoreType.DMA,
            pltpu.SemaphoreType.DMA,
            pltpu.SemaphoreType.DMA,
        ],
    )
    return f(Y4[0], Y4[1], src, dst)


def _update_body(a0_ref, a1_ref, x_ref, wua_ref, wux_ref, o_ref):
    h = lax.dot_general(x_ref[...], wux_ref[...], (((1,), (1,)), ((), ())),
                        preferred_element_type=jnp.float32)
    for c, a_ref in enumerate((a0_ref, a1_ref)):
        h = h + lax.dot_general(a_ref[...].astype(jnp.float32), wua_ref[c],
                                (((1,), (1,)), ((), ())),
                                preferred_element_type=jnp.float32)
    h = jnp.maximum(h, 0.0)
    m = jnp.max(h, axis=1, keepdims=True)
    s = jnp.sum(jnp.exp(h - m), axis=1, keepdims=True)
    o_ref[...] = h - m - jnp.log(s)


def _update(aggr4, x_pad, Wua4, Wux):
    return pl.pallas_call(
        _update_body,
        grid=(NP // RB,),
        in_specs=[pl.BlockSpec((RB, DCH), lambda i: (i, 0))] * NC + [
            pl.BlockSpec((RB, D_IN), lambda i: (i, 0)),
            pl.BlockSpec((NC, D_IN, DCH), lambda i: (0, 0, 0)),
            pl.BlockSpec((D_IN, D_IN), lambda i: (0, 0)),
        ],
        out_specs=pl.BlockSpec((RB, D_IN), lambda i: (i, 0)),
        out_shape=jax.ShapeDtypeStruct((NP, D_IN), jnp.float32),
    )(*aggr4, x_pad, Wua4, Wux)


def kernel(x, edge_index, W_lin, b_lin, W_up):
    src, dst = edge_index[0], edge_index[1]
    x_pad = jnp.pad(x, ((0, NP - N), (0, 0)))
    b2 = b_lin.reshape(1, D_OUT)
    Wua4 = W_up[:, :D_OUT].reshape(D_IN, NC, DCH).transpose(1, 0, 2)
    Wux = W_up[:, D_OUT:]

    Y4 = _msg_linear(x_pad, W_lin, b2)
    aggr4 = _segment_max(Y4, src, dst)
    out = _update(aggr4, x_pad, Wua4, Wux)
    return out[:N]


# speedup vs baseline: 1.0019x; 1.0019x over previous
import jax
import jax.numpy as jnp
from jax import lax
from jax.experimental import pallas as pl
from jax.experimental.pallas import tpu as pltpu
from jax.experimental.pallas import tpu_sc as plsc

N = 10000
E = 160000
D_IN = 256
D_OUT = 768
NC = 2
DCH = D_OUT // NC
NGRP = DCH // 32
NW = 32
NPW = 320
NP = NW * NPW
RB = 1024
EB = 2000
NB = E // EB
G = 64
GQ = G // 16
C_CAP = 8192
LIST_SZ = C_CAP + ((EB + 16 + G - 1) // G + 1) * G
DUMMY = NPW


def _lin_body(x_ref, w_ref, b_ref, *y_refs):
    y = lax.dot_general(x_ref[...], w_ref[...], (((1,), (1,)), ((), ())),
                        preferred_element_type=jnp.float32)
    y = jnp.maximum(y + b_ref[0, :], 0.0)
    for c in range(NC):
        y_refs[c][...] = y[:, c * DCH:(c + 1) * DCH].astype(jnp.bfloat16)


def _msg_linear(x_pad, W_lin, b2):
    return pl.pallas_call(
        _lin_body,
        grid=(NP // RB,),
        in_specs=[
            pl.BlockSpec((RB, D_IN), lambda i: (i, 0)),
            pl.BlockSpec((D_OUT, D_IN), lambda i: (0, 0)),
            pl.BlockSpec((1, D_OUT), lambda i: (0, 0)),
        ],
        out_specs=[pl.BlockSpec((RB, DCH), lambda i: (i, 0))] * NC,
        out_shape=[jax.ShapeDtypeStruct((NP, DCH), jnp.bfloat16)] * NC,
    )(x_pad, W_lin, b2)


def _segmax_body(y0, y1, src_hbm, dst_hbm,
                 o0, o1,
                 acc, lst, srcb0, dstb0, srcb1, dstb1, rb0, rb1, ix0, ix1,
                 sg0, sg1, ss0, ss1):
    wid = lax.axis_index("s") * 2 + lax.axis_index("c")
    base = wid * NPW
    iota = lax.broadcasted_iota(jnp.int32, (16,), 0)

    def zero_body(i, _):
        lst[pl.ds(i * 16, 16)] = jnp.full((16,), DUMMY, jnp.int32)
        return 0
    lax.fori_loop(0, LIST_SZ // 16, zero_body, 0)

    def stage_start(b, sb, db, sem):
        off = b * EB
        pltpu.make_async_copy(src_hbm.at[pl.ds(off, EB)], sb, sem).start()
        pltpu.make_async_copy(dst_hbm.at[pl.ds(off, EB)], db, sem).start()

    def stage_wait(b, sb, db, sem):
        off = b * EB
        pltpu.make_async_copy(src_hbm.at[pl.ds(off, EB)], sb, sem).wait()
        pltpu.make_async_copy(dst_hbm.at[pl.ds(off, EB)], db, sem).wait()

    def filt(sb, db, lbase, cnt):
        def fb(i, cn):
            d = db[pl.ds(i * 16, 16)]
            s = sb[pl.ds(i * 16, 16)]
            ldl = d - base
            m = (ldl >= 0) & (ldl < NPW)
            mi = jnp.where(m, 1, 0).astype(jnp.int32)
            incl = plsc.cumsum(mi)
            pos = (lbase + cn - 1) + incl
            plsc.store_scatter(lst, [pos], s * 512 + ldl, mask=m)
            return cn + incl[15]
        return lax.fori_loop(0, EB // 16, fb, cnt)

    def pad16(lbase, cnt):
        pos = lbase + cnt + iota
        plsc.store_scatter(lst, [pos], jnp.full((16,), DUMMY, jnp.int32))

    stage_start(0, srcb0, dstb0, ss0)

    def p1_body(p, carry):
        cnt, spill = carry

        def half(b, sb, db, sem, cnt, spill):
            newcnt = filt(sb, db, 0, cnt)
            ok = newcnt <= C_CAP
            cnt = jnp.where(ok, newcnt, cnt)
            spill = jnp.where((~ok) & (spill == NB), b, spill)
            return cnt, spill

        b0 = p * 2
        stage_start(b0 + 1, srcb1, dstb1, ss1)
        stage_wait(b0, srcb0, dstb0, ss0)
        cnt, spill = half(b0, srcb0, dstb0, ss0, cnt, spill)

        @pl.when(b0 + 2 < NB)
        def _():
            stage_start(b0 + 2, srcb0, dstb0, ss0)
        stage_wait(b0 + 1, srcb1, dstb1, ss1)
        cnt, spill = half(b0 + 1, srcb1, dstb1, ss1, cnt, spill)
        return cnt, spill

    cnt, spill = lax.fori_loop(0, NB // 2, p1_body,
                               (jnp.int32(0), jnp.int32(NB)))
    pad16(0, cnt)
    nq_main = (cnt + 15) // 16

    def run_waves(yc, ebase, nq):
        nw = (nq + GQ - 1) // GQ

        def issue(w, rbk, ixk, semk):
            def ub(i, _):
                pv = lst[pl.ds(ebase + w * G + i * 16, 16)]
                ixk[pl.ds(i * 16, 16)] = jnp.right_shift(pv, 9)
                return 0
            lax.fori_loop(0, G // 16, ub, 0)
            pltpu.make_async_copy(yc.at[ixk], rbk, semk).start()

        def wait_g(yc, rbk, ixk, semk):
            pltpu.make_async_copy(yc.at[ixk], rbk, semk).wait()

        def process(w, rbk):
            ne = jnp.minimum(nq - w * GQ, GQ) * 16

            def eb_body(j, _):
                pk = lst[pl.ds(ebase + w * G + j, 16)][0]
                ld = lax.rem(pk, 512)

                @plsc.parallel_loop(0, NGRP, unroll=NGRP)
                def cb(g, ld=ld, j=j):
                    sl = pl.ds(g * 32, 32)
                    acc[ld, sl] = jnp.maximum(acc[ld, sl], rbk[j, sl])
                return 0
            lax.fori_loop(0, ne, eb_body, 0)

        @pl.when(nw > 0)
        def _():
            issue(jnp.int32(0), rb0, ix0, sg0)

        def pair_body(p2, _):
            w0 = p2 * 2
            w1 = w0 + 1

            @pl.when(w1 < nw)
            def _():
                issue(w1, rb1, ix1, sg1)
            wait_g(yc, rb0, ix0, sg0)
            process(w0, rb0)

            @pl.when(w0 + 2 < nw)
            def _():
                issue(w0 + 2, rb0, ix0, sg0)

            @pl.when(w1 < nw)
            def _():
                wait_g(yc, rb1, ix1, sg1)
                process(w1, rb1)
            return 0
        lax.fori_loop(0, (nw + 1) // 2, pair_body, 0)

    for yc, oc in ((y0, o0), (y1, o1)):
        pltpu.sync_copy(yc.at[pl.ds(base, NPW)], acc.at[pl.ds(0, NPW)])
        run_waves(yc, 0, nq_main)

        def spill_body(b, _, yc=yc):
            off = b * EB
            pltpu.sync_copy(src_hbm.at[pl.ds(off, EB)], srcb0)
            pltpu.sync_copy(dst_hbm.at[pl.ds(off, EB)], dstb0)
            tcnt = filt(srcb0, dstb0, C_CAP, jnp.int32(0))
            pad16(C_CAP, tcnt)
            run_waves(yc, C_CAP, (tcnt + 15) // 16)
            return 0
        lax.fori_loop(spill, NB, spill_body, 0)

        pltpu.sync_copy(acc.at[pl.ds(0, NPW)], oc.at[pl.ds(base, NPW)])


def _segment_max(Y4, src, dst):
    mesh = plsc.VectorSubcoreMesh(core_axis_name="c", subcore_axis_name="s")
    f = pl.kernel(
        _segmax_body,
        out_type=[jax.ShapeDtypeStruct((NP, DCH), jnp.bfloat16)] * NC,
        mesh=mesh,
        compiler_params=pltpu.CompilerParams(use_tc_tiling_on_sc=False,
                                             needs_layout_passes=False),
        scratch_types=[
            pltpu.VMEM((NPW + 1, DCH), jnp.bfloat16),
            pltpu.VMEM((LIST_SZ,), jnp.int32),
            pltpu.VMEM((EB,), jnp.int32),
            pltpu.VMEM((EB,), jnp.int32),
            pltpu.VMEM((EB,), jnp.int32),
            pltpu.VMEM((EB,), jnp.int32),
            pltpu.VMEM((G, DCH), jnp.bfloat16),
            pltpu.VMEM((G, DCH), jnp.bfloat16),
            pltpu.VMEM((G,), jnp.int32),
            pltpu.VMEM((G,), jnp.int32),
            pltpu.SemaphoreType.DMA,
            pltpu.SemaphoreType.DMA,
            pltpu.SemaphoreType.DMA,
            pltpu.SemaphoreType.DMA,
        ],
    )
    return f(Y4[0], Y4[1], src, dst)


def _update_body(a0_ref, a1_ref, x_ref, wua_ref, wux_ref, o_ref):
    h = lax.dot_general(x_ref[...], wux_ref[...], (((1,), (1,)), ((), ())),
                        preferred_element_type=jnp.float32)
    for c, a_ref in enumerate((a0_ref, a1_ref)):
        h = h + lax.dot_general(a_ref[...].astype(jnp.float32), wua_ref[c],
                                (((1,), (1,)), ((), ())),
                                preferred_element_type=jnp.float32)
    h = jnp.maximum(h, 0.0)
    m = jnp.max(h, axis=1, keepdims=True)
    s = jnp.sum(jnp.exp(h - m), axis=1, keepdims=True)
    o_ref[...] = h - m - jnp.log(s)


def _update(aggr4, x_pad, Wua4, Wux):
    return pl.pallas_call(
        _update_body,
        grid=(NP // RB,),
        in_specs=[pl.BlockSpec((RB, DCH), lambda i: (i, 0))] * NC + [
            pl.BlockSpec((RB, D_IN), lambda i: (i, 0)),
            pl.BlockSpec((NC, D_IN, DCH), lambda i: (0, 0, 0)),
            pl.BlockSpec((D_IN, D_IN), lambda i: (0, 0)),
        ],
        out_specs=pl.BlockSpec((RB, D_IN), lambda i: (i, 0)),
        out_shape=jax.ShapeDtypeStruct((NP, D_IN), jnp.float32),
    )(*aggr4, x_pad, Wua4, Wux)


def kernel(x, edge_index, W_lin, b_lin, W_up):
    src, dst = edge_index[0], edge_index[1]
    x_pad = jnp.pad(x, ((0, NP - N), (0, 0)))
    b2 = b_lin.reshape(1, D_OUT)
    Wua4 = W_up[:, :D_OUT].reshape(D_IN, NC, DCH).transpose(1, 0, 2)
    Wux = W_up[:, D_OUT:]

    Y4 = _msg_linear(x_pad, W_lin, b2)
    aggr4 = _segment_max(Y4, src, dst)
    out = _update(aggr4, x_pad, Wua4, Wux)
    return out[:N]
